# baseline (speedup 1.0000x reference)
import functools

import jax
import jax.numpy as jnp
from jax import lax
from jax.experimental import pallas as pl
from jax.experimental.pallas import tpu as pltpu
from jax.experimental.pallas import tpu_sc as plsc

_BS, _K, _S, _D = 1024, 32, 8, 128
_AGG = 0.5
_NC, _NS = 2, 16
_NW = _NC * _NS
_BPW = _BS // _NW
_V = _D // 16
_R = _K * _S
_REG = _K + 1


def _sc_body(edge, masks, ent, selfv, sv_out, nv_out,
             ebuf, mbuf, entbuf, nvbuf, rdbuf, selfbuf, svbuf, idxbuf, zbuf,
             accsh, sem_e, sem_m, sem_t, sem_o, sem_a, sem_r):
    wid = lax.axis_index("c") * _NS + lax.axis_index("s")
    sid = lax.axis_index("s")
    b0 = wid * _BPW
    iota = lax.iota(jnp.int32, 16)
    ksel = lax.shift_right_logical(iota, 3)
    zero16 = jnp.zeros((16,), jnp.float32)
    one16 = jnp.ones((16,), jnp.int32)

    def zinit(r, _):
        for v in range(_V):
            zbuf[r, pl.ds(v * 16, 16)] = zero16
        return 0
    lax.fori_loop(0, _REG, zinit, 0)

    pltpu.sync_copy(selfv.at[pl.ds(b0, _BPW)], selfbuf)

    def start_in(j, p):
        bb = b0 + j
        pltpu.async_copy(edge.at[pl.ds(bb * _R, _R)], ebuf.at[p], sem_e)
        pltpu.async_copy(masks.at[bb], mbuf.at[p], sem_m)
        pltpu.async_copy(ent.at[pl.ds(bb * _K, _K)], entbuf.at[p], sem_t)

    def region(p):
        return sid * (2 * _REG) + _REG * p

    def prep(j, p):
        bb = b0 + j
        pltpu.make_async_copy(masks.at[bb], mbuf.at[p], sem_m).wait()
        base = region(p)
        basev = lax.broadcast(base, (16,))
        trash = lax.broadcast(base + _K, (16,))
        for c in range(16):
            mi = mbuf[p, pl.ds(c * 16, 16)].astype(jnp.int32)
            kvec = basev + (2 * c) + ksel
            idxbuf[p, c // 8, pl.ds((c % 8) * 16, 16)] = (
                mi * kvec + (one16 - mi) * trash)
        pltpu.sync_copy(zbuf, accsh.at[pl.ds(base, _REG)])
        pltpu.make_async_copy(edge.at[pl.ds(bb * _R, _R)], ebuf.at[p], sem_e).wait()
        for h in range(2):
            pltpu.async_copy(ebuf.at[p, pl.ds(h * 128, 128)],
                             accsh.at[idxbuf.at[p, h]], sem_a, add=True)

    start_in(0, 0)
    start_in(1, 1)
    prep(0, 0)

    def iter_body(i, _):
        p = lax.rem(i, 2)
        np_ = 1 - p
        bb = b0 + i

        for h in range(2):
            pltpu.make_async_copy(ebuf.at[p, pl.ds(h * 128, 128)],
                                  accsh.at[idxbuf.at[p, h]], sem_a).wait()
        pltpu.async_copy(accsh.at[pl.ds(region(p), _K)], rdbuf.at[p], sem_r)

        @pl.when(i + 1 < _BPW)
        def _():
            prep(i + 1, np_)

        @pl.when(i >= 2)
        def _():
            pltpu.make_async_copy(nvbuf.at[p], nv_out.at[pl.ds((bb - 2) * _K, _K)],
                                  sem_o).wait()

        pltpu.make_async_copy(accsh.at[pl.ds(region(p), _K)], rdbuf.at[p], sem_r).wait()
        pltpu.make_async_copy(ent.at[pl.ds(bb * _K, _K)], entbuf.at[p], sem_t).wait()

        def kk_body(kk, sv_acc):
            m16 = mbuf[p, pl.ds(kk * 16, 16)]
            for half in range(2):
                k = kk * 2 + half
                cnt = jnp.float32(0.0)
                for s in range(_S):
                    cnt = cnt + m16[half * _S + s]
                scale = (jnp.full((16,), _AGG, jnp.float32)
                         / jnp.maximum(lax.broadcast(cnt, (16,)), 1.0))
                out = []
                for v in range(_V):
                    nv_v = (entbuf[p, k, pl.ds(v * 16, 16)]
                            + scale * rdbuf[p, k, pl.ds(v * 16, 16)])
                    nvbuf[p, k, pl.ds(v * 16, 16)] = nv_v
                    out.append(sv_acc[v] + nv_v)
                sv_acc = tuple(out)
            return sv_acc

        sv0 = tuple(jnp.zeros((16,), jnp.float32) for _ in range(_V))
        sv = lax.fori_loop(0, _K // 2, kk_body, sv0)
        for v in range(_V):
            svbuf[i, pl.ds(v * 16, 16)] = (
                selfbuf[i, pl.ds(v * 16, 16)] + sv[v] * jnp.float32(_AGG / _K))

        pltpu.async_copy(nvbuf.at[p], nv_out.at[pl.ds(bb * _K, _K)], sem_o)

        @pl.when(i + 2 < _BPW)
        def _():
            start_in(i + 2, p)
        return 0

    lax.fori_loop(0, _BPW, iter_body, 0)

    for j in (_BPW - 2, _BPW - 1):
        pltpu.make_async_copy(
            nvbuf.at[lax.rem(jnp.int32(j), 2)],
            nv_out.at[pl.ds((b0 + j) * _K, _K)], sem_o).wait()

    pltpu.sync_copy(svbuf, sv_out.at[pl.ds(b0, _BPW)])


@functools.cache
def _build_sc_call():
    return functools.partial(
        pl.kernel,
        mesh=plsc.VectorSubcoreMesh(core_axis_name="c", subcore_axis_name="s"),
        out_type=[
            jax.ShapeDtypeStruct((_BS, _D), jnp.float32),
            jax.ShapeDtypeStruct((_BS * _K, _D), jnp.float32),
        ],
        scratch_types=[
            pltpu.VMEM((2, _R, _D), jnp.float32),
            pltpu.VMEM((2, _R), jnp.float32),
            pltpu.VMEM((2, _K, _D), jnp.float32),
            pltpu.VMEM((2, _K, _D), jnp.float32),
            pltpu.VMEM((2, _K, _D), jnp.float32),
            pltpu.VMEM((_BPW, _D), jnp.float32),
            pltpu.VMEM((_BPW, _D), jnp.float32),
            pltpu.VMEM((2, 2, 128), jnp.int32),
            pltpu.VMEM((_REG, _D), jnp.float32),
            pltpu.VMEM_SHARED((_NS * 2 * _REG, _D), jnp.float32),
            pltpu.SemaphoreType.DMA,
            pltpu.SemaphoreType.DMA,
            pltpu.SemaphoreType.DMA,
            pltpu.SemaphoreType.DMA,
            pltpu.SemaphoreType.DMA,
            pltpu.SemaphoreType.DMA,
        ],
    )(_sc_body)


def kernel(self_vectors, neighbor_entity_vectors, neighbor_edge_vectors, masks, W, b):
    del W, b
    bs = self_vectors.shape[0]
    sv, nv = _build_sc_call()(
        neighbor_edge_vectors.reshape(_BS * _K * _S, _D),
        masks.reshape(_BS, _R),
        neighbor_entity_vectors.reshape(_BS * _K, _D),
        self_vectors.reshape(_BS, _D))
    return (sv.reshape(bs, -1, _D), nv.reshape(_BS, 1, _K, _D))


# speedup vs baseline: 1.6119x; 1.6119x over previous
import functools

import jax
import jax.numpy as jnp
from jax import lax
from jax.experimental import pallas as pl
from jax.experimental.pallas import tpu as pltpu
from jax.experimental.pallas import tpu_sc as plsc

_BS, _K, _S, _D = 1024, 32, 8, 128
_AGG = 0.5
_NC, _NS = 2, 16
_NW = _NC * _NS
_BPW = _BS // _NW
_V = _D // 16


def _sc_body(edge, masks, ent, selfv, sv_out, nv_out,
             ebuf, mbuf, entbuf, nvbuf, selfbuf, svbuf,
             sem_e, sem_s, sem_o):
    wid = lax.axis_index("c") * _NS + lax.axis_index("s")
    b0 = wid * _BPW

    pltpu.sync_copy(selfv.at[pl.ds(b0, _BPW)], selfbuf)

    def start_in(j, sl):
        bb = b0 + j
        pltpu.async_copy(edge.at[pl.ds(bb * (_K * _S), _K * _S)], ebuf.at[sl], sem_e)
        pltpu.async_copy(masks.at[bb], mbuf.at[sl], sem_s)
        pltpu.async_copy(ent.at[pl.ds(bb * _K, _K)], entbuf.at[sl], sem_s)

    def wait_in(j, sl):
        bb = b0 + j
        pltpu.make_async_copy(edge.at[pl.ds(bb * (_K * _S), _K * _S)], ebuf.at[sl], sem_e).wait()
        pltpu.make_async_copy(masks.at[bb], mbuf.at[sl], sem_s).wait()
        pltpu.make_async_copy(ent.at[pl.ds(bb * _K, _K)], entbuf.at[sl], sem_s).wait()

    start_in(0, 0)

    def iter_body(i, _):
        sl = lax.rem(i, 2)
        nsl = 1 - sl
        bb = b0 + i

        @pl.when(i + 1 < _BPW)
        def _():
            start_in(i + 1, nsl)

        @pl.when(i >= 2)
        def _():
            pltpu.make_async_copy(nvbuf.at[sl], nv_out.at[pl.ds((bb - 2) * _K, _K)], sem_o).wait()

        wait_in(i, sl)

        def kk_body(kk, sv_acc):
            m16 = mbuf[sl, pl.ds(kk * 16, 16)]
            for half in range(2):
                k = kk * 2 + half
                cnt = jnp.float32(0.0)
                accs = [jnp.zeros((16,), jnp.float32)] * _V
                for s in range(_S):
                    lane = half * _S + s
                    r = k * _S + s
                    m = m16[lane]
                    cnt = cnt + m
                    mvec = lax.broadcast(m, (16,))
                    for v in range(_V):
                        accs[v] = accs[v] + mvec * ebuf[sl, r, pl.ds(v * 16, 16)]
                scale = (jnp.full((16,), _AGG, jnp.float32)
                         / jnp.maximum(lax.broadcast(cnt, (16,)), 1.0))
                out = []
                for v in range(_V):
                    nv_v = entbuf[sl, k, pl.ds(v * 16, 16)] + scale * accs[v]
                    nvbuf[sl, k, pl.ds(v * 16, 16)] = nv_v
                    out.append(sv_acc[v] + nv_v)
                sv_acc = tuple(out)
            return sv_acc

        sv0 = tuple(jnp.zeros((16,), jnp.float32) for _ in range(_V))
        sv = lax.fori_loop(0, _K // 2, kk_body, sv0, unroll=2)
        for v in range(_V):
            svbuf[i, pl.ds(v * 16, 16)] = (
                selfbuf[i, pl.ds(v * 16, 16)] + sv[v] * jnp.float32(_AGG / _K))

        pltpu.async_copy(nvbuf.at[sl], nv_out.at[pl.ds(bb * _K, _K)], sem_o)
        return 0

    lax.fori_loop(0, _BPW, iter_body, 0)

    for j in (_BPW - 2, _BPW - 1):
        pltpu.make_async_copy(
            nvbuf.at[lax.rem(jnp.int32(j), 2)],
            nv_out.at[pl.ds((b0 + j) * _K, _K)], sem_o).wait()

    pltpu.sync_copy(svbuf, sv_out.at[pl.ds(b0, _BPW)])


@functools.cache
def _build_sc_call():
    return functools.partial(
        pl.kernel,
        mesh=plsc.VectorSubcoreMesh(core_axis_name="c", subcore_axis_name="s"),
        out_type=[
            jax.ShapeDtypeStruct((_BS, _D), jnp.float32),
            jax.ShapeDtypeStruct((_BS * _K, _D), jnp.float32),
        ],
        scratch_types=[
            pltpu.VMEM((2, _K * _S, _D), jnp.float32),
            pltpu.VMEM((2, _K * _S), jnp.float32),
            pltpu.VMEM((2, _K, _D), jnp.float32),
            pltpu.VMEM((2, _K, _D), jnp.float32),
            pltpu.VMEM((_BPW, _D), jnp.float32),
            pltpu.VMEM((_BPW, _D), jnp.float32),
            pltpu.SemaphoreType.DMA,
            pltpu.SemaphoreType.DMA,
            pltpu.SemaphoreType.DMA,
        ],
    )(_sc_body)


def kernel(self_vectors, neighbor_entity_vectors, neighbor_edge_vectors, masks, W, b):
    del W, b
    bs = self_vectors.shape[0]
    edge2 = neighbor_edge_vectors.reshape(_BS * _K * _S, _D)
    masks2 = masks.reshape(_BS, _K * _S)
    ent2 = neighbor_entity_vectors.reshape(_BS * _K, _D)
    self2 = self_vectors.reshape(_BS, _D)
    sv, nv = _build_sc_call()(edge2, masks2, ent2, self2)
    return (sv.reshape(bs, -1, _D), nv.reshape(_BS, 1, _K, _D))
